# Initial kernel scaffold; baseline (speedup 1.0000x reference)
import jax
import jax.numpy as jnp
from jax.experimental import pallas as pl


def kernel(x, W, b):
    raise NotImplementedError("write your pallas kernel here")


# speedup vs baseline: 35.5914x; 35.5914x over previous
import jax
import jax.numpy as jnp
from jax.experimental import pallas as pl
from jax.experimental.pallas import tpu as pltpu

_B, _C, _IN = 128, 128, 500
_N = 512


def _body(x_ref, s_ref, out_ref, probs_t_ref):
    xb = x_ref[0]
    s_row = s_ref[0]

    s_mat = jnp.broadcast_to(s_row, (_N, _N))
    s_col_mat = s_mat.T

    gt = (s_mat > s_col_mat).astype(jnp.float32)
    eq = (s_mat == s_col_mat).astype(jnp.float32)
    a_idx = jax.lax.broadcasted_iota(jnp.int32, (_N, _N), 1).astype(jnp.float32)
    j_idx = jax.lax.broadcasted_iota(jnp.int32, (_N, _N), 0).astype(jnp.float32)
    tie = eq * (a_idx < j_idx).astype(jnp.float32)

    rank = jnp.sum(gt + tie, axis=1, keepdims=True)
    dupf = (jnp.sum(eq, axis=1, keepdims=True) >= 2.0).astype(jnp.float32)

    m = (rank == a_idx).astype(jnp.float32)

    is_dup_row = jnp.sum(m * dupf, axis=0, keepdims=True)
    nan_row = is_dup_row > 0.0

    nan = jnp.float32(jnp.nan)
    probs_t_ref[0] = jnp.where(nan_row, nan, m)

    gathered = jnp.dot(xb, m, preferred_element_type=jnp.float32)
    out_ref[0] = jnp.where(nan_row, nan, gathered)


def kernel(x, W, b):
    xp = jnp.pad(x, ((0, 0), (0, 0), (0, _N - _IN)))
    score = jnp.einsum('bcn,c->bn', xp, W) + b
    s3 = score.reshape(_B, 1, _N)
    out, probs_t = pl.pallas_call(
        _body,
        grid=(_B,),
        in_specs=[
            pl.BlockSpec((1, _C, _N), lambda i: (i, 0, 0)),
            pl.BlockSpec((1, 1, _N), lambda i: (i, 0, 0)),
        ],
        out_specs=[
            pl.BlockSpec((1, _C, _N), lambda i: (i, 0, 0)),
            pl.BlockSpec((1, _N, _N), lambda i: (i, 0, 0)),
        ],
        out_shape=[
            jax.ShapeDtypeStruct((_B, _C, _N), jnp.float32),
            jax.ShapeDtypeStruct((_B, _N, _N), jnp.float32),
        ],
    )(xp, s3)
    return out, probs_t


# speedup vs baseline: 44.2486x; 1.2432x over previous
import jax
import jax.numpy as jnp
from jax.experimental import pallas as pl
from jax.experimental.pallas import tpu as pltpu

_B, _C, _IN = 128, 128, 500
_N = 512


def _body(x_ref, w_ref, b_ref, out_ref, probs_t_ref):
    xb = x_ref[0]
    bias = b_ref[0, 0]

    s500 = jnp.dot(w_ref[...], xb, preferred_element_type=jnp.float32) + bias
    s_row = jnp.concatenate(
        [s500, jnp.full((1, _N - _IN), bias, jnp.float32)], axis=1)

    s_mat = jnp.broadcast_to(s_row, (_N, _N))
    s_col_mat = s_mat.T

    gt = (s_mat > s_col_mat).astype(jnp.float32)
    eq = (s_mat == s_col_mat).astype(jnp.float32)
    a_idx = jax.lax.broadcasted_iota(jnp.int32, (_N, _N), 1).astype(jnp.float32)
    j_idx = jax.lax.broadcasted_iota(jnp.int32, (_N, _N), 0).astype(jnp.float32)
    tie = eq * (a_idx < j_idx).astype(jnp.float32)

    rank = jnp.sum(gt + tie, axis=1, keepdims=True)
    dupf = (jnp.sum(eq, axis=1, keepdims=True) >= 2.0).astype(jnp.float32)

    m = (rank == a_idx).astype(jnp.float32)

    is_dup_row = jnp.sum(m * dupf, axis=0, keepdims=True)
    nan_row = is_dup_row > 0.0

    nan = jnp.float32(jnp.nan)
    probs_t_ref[0] = jnp.where(nan_row, nan, m)

    gathered = jnp.dot(xb, m[:_IN, :], preferred_element_type=jnp.float32)
    out_ref[0] = jnp.where(nan_row, nan, gathered)


def kernel(x, W, b):
    w2 = W.reshape(1, _C)
    b2 = jnp.reshape(b, (1, 1)).astype(jnp.float32)
    out, probs_t = pl.pallas_call(
        _body,
        grid=(_B,),
        in_specs=[
            pl.BlockSpec((1, _C, _IN), lambda i: (i, 0, 0)),
            pl.BlockSpec((1, _C), lambda i: (0, 0)),
            pl.BlockSpec(memory_space=pltpu.SMEM),
        ],
        out_specs=[
            pl.BlockSpec((1, _C, _N), lambda i: (i, 0, 0)),
            pl.BlockSpec((1, _N, _N), lambda i: (i, 0, 0)),
        ],
        out_shape=[
            jax.ShapeDtypeStruct((_B, _C, _N), jnp.float32),
            jax.ShapeDtypeStruct((_B, _N, _N), jnp.float32),
        ],
    )(x, w2, b2)
    return out, probs_t


# speedup vs baseline: 52.7450x; 1.1920x over previous
import jax
import jax.numpy as jnp
from jax.experimental import pallas as pl
from jax.experimental.pallas import tpu as pltpu

_B, _C, _IN = 128, 128, 500
_N = 512


_BBLK = 2


def _body(x_ref, w_ref, b_ref, out_ref, probs_t_ref):
    bias = b_ref[0, 0]
    for t in range(_BBLK):
        _one_batch(t, x_ref, w_ref, bias, out_ref, probs_t_ref)


def _one_batch(t, x_ref, w_ref, bias, out_ref, probs_t_ref):
    xb = x_ref[t]

    s500 = jnp.dot(w_ref[...], xb, preferred_element_type=jnp.float32) + bias
    s_row = jnp.concatenate(
        [s500, jnp.full((1, _N - _IN), bias, jnp.float32)], axis=1)

    s_mat = jnp.broadcast_to(s_row, (_N, _N))
    s_col_mat = s_mat.T

    gt = (s_mat > s_col_mat).astype(jnp.float32)
    eq = (s_mat == s_col_mat).astype(jnp.float32)
    a_idx = jax.lax.broadcasted_iota(jnp.int32, (_N, _N), 1).astype(jnp.float32)
    j_idx = jax.lax.broadcasted_iota(jnp.int32, (_N, _N), 0).astype(jnp.float32)
    tie = eq * (a_idx < j_idx).astype(jnp.float32)

    rank = jnp.sum(gt + tie, axis=1, keepdims=True)
    dupf = (jnp.sum(eq, axis=1, keepdims=True) >= 2.0).astype(jnp.float32)

    m = (rank == a_idx).astype(jnp.float32)

    is_dup_row = jnp.sum(m * dupf, axis=0, keepdims=True)
    nan_row = is_dup_row > 0.0

    nan = jnp.float32(jnp.nan)
    probs_t_ref[t] = jnp.where(nan_row, nan, m)

    gathered = jnp.dot(xb, m[:_IN, :], preferred_element_type=jnp.float32)
    out_ref[t] = jnp.where(nan_row, nan, gathered)


def kernel(x, W, b):
    w2 = W.reshape(1, _C)
    b2 = jnp.reshape(b, (1, 1)).astype(jnp.float32)
    out, probs_t = pl.pallas_call(
        _body,
        grid=(_B // _BBLK,),
        in_specs=[
            pl.BlockSpec((_BBLK, _C, _IN), lambda i: (i, 0, 0)),
            pl.BlockSpec((1, _C), lambda i: (0, 0)),
            pl.BlockSpec(memory_space=pltpu.SMEM),
        ],
        out_specs=[
            pl.BlockSpec((_BBLK, _C, _N), lambda i: (i, 0, 0)),
            pl.BlockSpec((_BBLK, _N, _N), lambda i: (i, 0, 0)),
        ],
        out_shape=[
            jax.ShapeDtypeStruct((_B, _C, _N), jnp.float32),
            jax.ShapeDtypeStruct((_B, _N, _N), jnp.float32),
        ],
    )(x, w2, b2)
    return out, probs_t


# speedup vs baseline: 56.6933x; 1.0749x over previous
import jax
import jax.numpy as jnp
from jax.experimental import pallas as pl
from jax.experimental.pallas import tpu as pltpu

_B, _C, _IN = 128, 128, 500
_N = 512


_BBLK = 4


def _body(x_ref, w_ref, b_ref, out_ref, probs_t_ref):
    bias = b_ref[0, 0]
    for t in range(_BBLK):
        _one_batch(t, x_ref, w_ref, bias, out_ref, probs_t_ref)


def _one_batch(t, x_ref, w_ref, bias, out_ref, probs_t_ref):
    xb = x_ref[t]

    s500 = jnp.dot(w_ref[...], xb, preferred_element_type=jnp.float32) + bias
    s_row = jnp.concatenate(
        [s500, jnp.full((1, _N - _IN), bias, jnp.float32)], axis=1)

    s_mat = jnp.broadcast_to(s_row, (_N, _N))
    s_col_mat = s_mat.T

    gt = (s_mat > s_col_mat).astype(jnp.float32)
    eq = (s_mat == s_col_mat).astype(jnp.float32)
    a_idx = jax.lax.broadcasted_iota(jnp.int32, (_N, _N), 1).astype(jnp.float32)
    j_idx = jax.lax.broadcasted_iota(jnp.int32, (_N, _N), 0).astype(jnp.float32)
    tie = eq * (a_idx < j_idx).astype(jnp.float32)

    rank = jnp.sum(gt + tie, axis=1, keepdims=True)
    dupf = (jnp.sum(eq, axis=1, keepdims=True) >= 2.0).astype(jnp.float32)

    m = (rank == a_idx).astype(jnp.float32)

    is_dup_row = jnp.sum(m * dupf, axis=0, keepdims=True)
    nan_row = is_dup_row > 0.0

    nan = jnp.float32(jnp.nan)
    probs_t_ref[t] = jnp.where(nan_row, nan, m)

    gathered = jnp.dot(xb, m[:_IN, :], preferred_element_type=jnp.float32)
    out_ref[t] = jnp.where(nan_row, nan, gathered)


def kernel(x, W, b):
    w2 = W.reshape(1, _C)
    b2 = jnp.reshape(b, (1, 1)).astype(jnp.float32)
    out, probs_t = pl.pallas_call(
        _body,
        grid=(_B // _BBLK,),
        in_specs=[
            pl.BlockSpec((_BBLK, _C, _IN), lambda i: (i, 0, 0)),
            pl.BlockSpec((1, _C), lambda i: (0, 0)),
            pl.BlockSpec(memory_space=pltpu.SMEM),
        ],
        out_specs=[
            pl.BlockSpec((_BBLK, _C, _N), lambda i: (i, 0, 0)),
            pl.BlockSpec((_BBLK, _N, _N), lambda i: (i, 0, 0)),
        ],
        out_shape=[
            jax.ShapeDtypeStruct((_B, _C, _N), jnp.float32),
            jax.ShapeDtypeStruct((_B, _N, _N), jnp.float32),
        ],
    )(x, w2, b2)
    return out, probs_t
